# Initial kernel scaffold; baseline (speedup 1.0000x reference)
import jax
import jax.numpy as jnp
from jax.experimental import pallas as pl


def kernel(input, token_table, prefix_table):
    raise NotImplementedError("write your pallas kernel here")


# speedup vs baseline: 3.0872x; 3.0872x over previous
import dataclasses
import functools

import jax
import jax.numpy as jnp
from jax import lax
from jax.experimental import pallas as pl
from jax.experimental.pallas import tpu as pltpu
from jax.experimental.pallas import tpu_sc as plsc

_NUM_EMB = 1000000
_D = 32
_PFX_ROWS = 41
_L = 16
_NC = 2
_NS = 16
_NW = _NC * _NS
_CH = 1024
_GW = 128


def _sc_lookup(inp_flat, token_table, prefix_table, n_rows):
    per_w = n_rows // _NW
    n_chunks = per_w // _CH
    mesh = plsc.VectorSubcoreMesh(core_axis_name="c", subcore_axis_name="s")
    cp = pltpu.CompilerParams()
    if "needs_layout_passes" in pltpu.CompilerParams.__dataclass_fields__:
        cp = dataclasses.replace(cp, needs_layout_passes=False)
    cp = dataclasses.replace(cp, use_tc_tiling_on_sc=False)

    @functools.partial(
        pl.kernel,
        compiler_params=cp,
        out_type=jax.ShapeDtypeStruct((n_rows, _D), jnp.float32),
        mesh=mesh,
        scratch_types=[
            pltpu.VMEM((_CH,), jnp.int32),
            pltpu.VMEM((_CH,), jnp.int32),
            pltpu.VMEM((_CH, _D), jnp.float32),
            pltpu.VMEM((_PFX_ROWS, _D), jnp.float32),
            pltpu.SemaphoreType.DMA,
        ],
    )
    def k(inp_hbm, tok_hbm, pfx_hbm, out_hbm, ids_v, tid_v, rows_v, pfx_v, sem):
        wid = lax.axis_index("s") * _NC + lax.axis_index("c")
        base = wid * per_w
        pltpu.sync_copy(pfx_hbm, pfx_v)

        @pl.loop(0, n_chunks)
        def _chunk(g):
            off = base + g * _CH
            pltpu.sync_copy(inp_hbm.at[pl.ds(off, _CH)], ids_v)

            @pl.loop(0, _CH // _L)
            def _adjust(q):
                v = ids_v[pl.ds(q * _L, _L)]
                is_pfx = v >= _NUM_EMB
                tid_v[pl.ds(q * _L, _L)] = jnp.where(is_pfx, 0, v)

            copies = [
                pltpu.async_copy(
                    tok_hbm.at[tid_v.at[pl.ds(j * _GW, _GW)]],
                    rows_v.at[pl.ds(j * _GW, _GW)],
                    sem,
                )
                for j in range(_CH // _GW)
            ]
            for cp in copies:
                cp.wait()

            @pl.loop(0, _CH // _L)
            def _fixup(q):
                v = ids_v[pl.ds(q * _L, _L)]
                is_pfx = v >= _NUM_EMB

                @pl.when(jnp.any(is_pfx))
                def _():
                    pidx = jnp.where(is_pfx, v - (_NUM_EMB - 1), 0)
                    rowids = lax.iota(jnp.int32, _L) + q * _L
                    for col in range(_D):
                        cols = jnp.full((_L,), col, jnp.int32)
                        vals = plsc.load_gather(pfx_v, [pidx, cols])
                        plsc.addupdate_scatter(
                            rows_v, [rowids, cols], vals, mask=is_pfx
                        )

            pltpu.sync_copy(rows_v, out_hbm.at[pl.ds(off, _CH)])

    return k(inp_flat, token_table, prefix_table)


def kernel(input, token_table, prefix_table):
    b, s = input.shape
    n = b * s
    inp_flat = input.reshape(n).astype(jnp.int32)
    out = _sc_lookup(inp_flat, token_table, prefix_table, n)
    return out.reshape(b, s, _D)


# speedup vs baseline: 12.0136x; 3.8914x over previous
import dataclasses
import functools

import jax
import jax.numpy as jnp
from jax import lax
from jax.experimental import pallas as pl
from jax.experimental.pallas import tpu as pltpu
from jax.experimental.pallas import tpu_sc as plsc

_NUM_EMB = 1000000
_D = 32
_PFX_ROWS = 41
_L = 16
_NC = 2
_NS = 16
_NW = _NC * _NS
_CH = 1024
_GW = 128


def _sc_relayout_table(tt_T):
    n_ch, n_tok = tt_T.shape
    CK = 512
    n_full = n_tok // CK
    per_w = n_full // _NW
    extra = n_full - per_w * _NW

    mesh = plsc.VectorSubcoreMesh(core_axis_name="c", subcore_axis_name="s")
    cp = pltpu.CompilerParams()
    if "needs_layout_passes" in pltpu.CompilerParams.__dataclass_fields__:
        cp = dataclasses.replace(cp, needs_layout_passes=False)
    cp = dataclasses.replace(cp, use_tc_tiling_on_sc=True)

    NT = CK // 128

    @functools.partial(
        pl.kernel,
        compiler_params=cp,
        out_type=jax.ShapeDtypeStruct((n_ch * n_full * CK,), jnp.float32),
        mesh=mesh,
        scratch_types=[
            pltpu.VMEM((2 * 4 * NT * 8, 128), jnp.float32),
            pltpu.VMEM((2 * CK * _D,), jnp.float32),
            pltpu.VMEM((CK * 33,), jnp.float32),
            pltpu.SemaphoreType.DMA,
            pltpu.SemaphoreType.DMA,
        ],
    )
    def k(tt_hbm, out_hbm, inb, rows, stage, isem, osem):
        lanes16 = lax.iota(jnp.int32, _L)
        wid = lax.axis_index("s") * _NC + lax.axis_index("c")
        cnt = jnp.where(wid == 0, per_w + extra, per_w)
        start = wid * per_w + jnp.minimum(wid, extra)

        def tile_base(buf, ci, j):
            return ((buf * 4 + ci) * NT + j) * 8

        def fire_in(chunk, buf):
            for ci in range(4):
                for j in range(NT):
                    pltpu.async_copy(
                        tt_hbm.at[pl.ds(8 * ci, 8), pl.ds(chunk * CK + 128 * j, 128)],
                        inb.at[pl.ds(tile_base(buf, ci, j), 8)],
                        isem,
                    )

        def drain_in(buf):
            for ci in range(4):
                for j in range(NT):
                    pltpu.make_async_copy(
                        tt_hbm.at[pl.ds(0, 8), pl.ds(0, 128)],
                        inb.at[pl.ds(tile_base(buf, ci, j), 8)],
                        isem,
                    ).wait()

        def drain_out(buf):
            pltpu.make_async_copy(
                out_hbm.at[pl.ds(0, CK * _D)],
                rows.at[pl.ds(buf * CK * _D, CK * _D)],
                osem,
            ).wait()

        def transpose(buf, n_groups):
            lanes33 = lanes16 * 33

            @plsc.parallel_loop(0, 32, unroll=2)
            def _c(c):
                ci = c >> 3
                kk = c & 7
                rowb = tile_base(buf, ci, 0) + kk
                for m in range(n_groups):
                    j = m >> 3
                    m2 = m & 7
                    v = inb[rowb + 8 * j, pl.ds(m2 * _L, _L)]
                    offs = lanes33 + ((j * 128 + m2 * _L) * 33 + c)
                    plsc.store_scatter(stage, [offs], v)

            @plsc.parallel_loop(0, n_groups, unroll=2)
            def _g(g):
                for u in range(_L):
                    l = g * _L + u
                    a0 = lanes16 + l * 33
                    v0 = plsc.load_gather(stage, [a0])
                    rows[pl.ds(buf * (CK * _D) + l * _D, _L)] = v0
                    v1 = plsc.load_gather(stage, [a0 + _L])
                    rows[pl.ds(buf * (CK * _D) + l * _D + _L, _L)] = v1

        fire_in(start, 0)

        @pl.loop(0, per_w + extra)
        def _g(i):
            @pl.when(i < cnt)
            def _():
                cur = start + i
                buf = i & 1

                @pl.when(i + 1 < cnt)
                def _():
                    fire_in(cur + 1, 1 - buf)

                drain_in(buf)

                @pl.when(i >= 2)
                def _():
                    drain_out(buf)

                transpose(buf, CK // _L)
                pltpu.async_copy(
                    rows.at[pl.ds(buf * CK * _D, CK * _D)],
                    out_hbm.at[pl.ds(cur * CK * _D, CK * _D)],
                    osem,
                )

        @pl.when(cnt >= 2)
        def _():
            drain_out((cnt - 2) & 1)

        @pl.when(cnt >= 1)
        def _():
            drain_out((cnt - 1) & 1)

    return k(tt_T)


def _sc_lookup(inp_sflat, token_table, aux_table, n_seq, n_batch):
    cut = token_table.shape[0]
    n_aux = aux_table.shape[0]
    BK = 512
    n_blocks = n_seq * n_batch // BK
    per_w = n_blocks // _NW
    SPS = n_batch // BK
    TPB = n_batch // 128
    SLAB = 4 * TPB * 1024

    mesh = plsc.VectorSubcoreMesh(core_axis_name="c", subcore_axis_name="s")
    cp = pltpu.CompilerParams()
    if "needs_layout_passes" in pltpu.CompilerParams.__dataclass_fields__:
        cp = dataclasses.replace(cp, needs_layout_passes=False)
    cp = dataclasses.replace(cp, use_tc_tiling_on_sc=False)

    @functools.partial(
        pl.kernel,
        compiler_params=cp,
        out_type=jax.ShapeDtypeStruct((n_seq * SLAB,), jnp.float32),
        mesh=mesh,
        scratch_types=[
            pltpu.VMEM((per_w * BK,), jnp.int32),
            pltpu.VMEM((2 * BK,), jnp.int32),
            pltpu.VMEM((2 * BK, _D), jnp.float32),
            pltpu.VMEM((2 * 4 * 4096,), jnp.float32),
            pltpu.VMEM((BK * 33,), jnp.float32),
            pltpu.VMEM((n_aux, _D), jnp.float32),
            pltpu.SemaphoreType.DMA,
            pltpu.SemaphoreType.DMA,
        ],
    )
    def k(inp_hbm, tok_hbm, aux_hbm, out_hbm, ids_v, tid_v, rows_v, tiles_v,
          stage, aux_v, gsem, osem):
        lanes16 = lax.iota(jnp.int32, _L)
        lanes33 = lanes16 * 33
        wid = lax.axis_index("s") * _NC + lax.axis_index("c")
        start = wid * per_w
        pltpu.sync_copy(aux_hbm, aux_v)
        pltpu.sync_copy(inp_hbm.at[pl.ds(start * BK, per_w * BK)], ids_v)

        def fire_gather(i, buf):
            @plsc.parallel_loop(0, BK // (_L * 8), unroll=2)
            def _adj(g):
                for u in range(8):
                    o = g * (_L * 8) + u * _L
                    v = ids_v[pl.ds(i * BK + o, _L)]
                    tid_v[pl.ds(buf * BK + o, _L)] = jnp.where(v >= cut, 0, v)

            for w in range(BK // 128):
                pltpu.async_copy(
                    tok_hbm.at[tid_v.at[pl.ds(buf * BK + w * 128, 128)]],
                    rows_v.at[pl.ds(buf * BK + w * 128, 128)],
                    gsem,
                )

        def wait_gather(buf):
            for w in range(BK // 128):
                pltpu.make_async_copy(
                    tok_hbm.at[tid_v.at[pl.ds(0, 128)]],
                    rows_v.at[pl.ds(buf * BK + w * 128, 128)],
                    gsem,
                ).wait()

        def drain_out(buf):
            for ci in range(4):
                pltpu.make_async_copy(
                    out_hbm.at[pl.ds(0, 4096)],
                    tiles_v.at[pl.ds(buf * 16384 + ci * 4096, 4096)],
                    osem,
                ).wait()

        fire_gather(0, 0)

        @pl.loop(0, per_w)
        def _blk(i):
            buf = i & 1
            q = start + i
            s = q // SPS
            bj = q - s * SPS

            @pl.when(i + 1 < per_w)
            def _():
                fire_gather(i + 1, 1 - buf)

            wait_gather(buf)

            @pl.loop(0, BK // _L)
            def _fixup(g):
                v = ids_v[pl.ds(i * BK + g * _L, _L)]
                is_aux = v >= cut

                @pl.when(jnp.any(is_aux))
                def _():
                    aidx = jnp.where(
                        is_aux, v - cut + (v >= _NUM_EMB).astype(jnp.int32), 0
                    )
                    rowids = buf * BK + g * _L + lanes16
                    for col in range(_D):
                        cols = jnp.full((_L,), col, jnp.int32)
                        vals = plsc.load_gather(aux_v, [aidx, cols])
                        plsc.addupdate_scatter(
                            rows_v, [rowids, cols], vals, mask=is_aux
                        )

            @pl.when(i >= 2)
            def _():
                drain_out(buf)

            @plsc.parallel_loop(0, BK // _L, unroll=2)
            def _tr1(g):
                for u in range(_L):
                    l = g * _L + u
                    offs = lanes16 + l * 33
                    v0 = rows_v[buf * BK + l, pl.ds(0, _L)]
                    plsc.store_scatter(stage, [offs], v0)
                    v1 = rows_v[buf * BK + l, pl.ds(_L, _L)]
                    plsc.store_scatter(stage, [offs + _L], v1)

            @plsc.parallel_loop(0, BK // _L, unroll=2)
            def _tr2(g):
                bsub = g >> 3
                g2 = g & 7
                for c in range(_D):
                    ci = c >> 3
                    kk = c & 7
                    v = plsc.load_gather(stage, [lanes33 + (g * _L * 33 + c)])
                    tiles_v[
                        pl.ds(
                            buf * 16384 + ci * 4096 + bsub * 1024
                            + kk * 128 + g2 * _L,
                            _L,
                        )
                    ] = v

            for ci in range(4):
                pltpu.async_copy(
                    tiles_v.at[pl.ds(buf * 16384 + ci * 4096, 4096)],
                    out_hbm.at[
                        pl.ds(s * SLAB + ci * (TPB * 1024) + bj * 4096, 4096)
                    ],
                    osem,
                )

        @pl.when(per_w >= 2)
        def _():
            drain_out(per_w & 1)

        drain_out((per_w - 1) & 1)

    return k(inp_sflat, token_table, aux_table)


def kernel(input, token_table, prefix_table):
    b, s = input.shape
    cut = (token_table.shape[0] // 512) * 512
    inp_sflat = input.T.reshape(b * s).astype(jnp.int32)
    tbl_lin = _sc_relayout_table(token_table.T)
    tbl = tbl_lin.reshape(cut, _D)
    aux = jnp.concatenate(
        [jax.lax.slice(token_table, (cut, 0), (token_table.shape[0], _D)),
         prefix_table], axis=0)
    out_flat = _sc_lookup(inp_sflat, tbl, aux, s, b)
    out = out_flat.reshape(s, 4, b // 128, 8, 128).transpose(2, 4, 0, 1, 3)
    return out.reshape(b, s, _D)
